# Initial kernel scaffold; baseline (speedup 1.0000x reference)
import jax
import jax.numpy as jnp
from jax.experimental import pallas as pl


def kernel(x, beta, matrix, input, target):
    raise NotImplementedError("write your pallas kernel here")


# speedup vs baseline: 2.9036x; 2.9036x over previous
import jax
import jax.numpy as jnp
from jax.experimental import pallas as pl
from jax.experimental.pallas import tpu as pltpu

Q_MIN = 0.1
SUPRESSION = 0.1
COND_WEIGHT = 1.0

_B, _NOBJ, _H, _W = 8, 8, 128, 128
_N = _H * _W


def _atanh(v):
    return jnp.log1p(2.0 * v / (1.0 - v)) / 2.0


def _loss_kernel(x0_ref, x1_ref, beta_ref, m_ref, inp_ref, tgt_ref, out_ref):
    beta = beta_ref[0]
    x0 = x0_ref[0]
    x1 = x1_ref[0]
    inp = inp_ref[0, 0]
    tgt = tgt_ref[0, 0]

    q = _atanh(beta) ** 2 + Q_MIN
    loss_elem = (inp - tgt) ** 2

    msum = m_ref[0, 0]
    for i in range(1, _NOBJ):
        msum = msum + m_ref[0, i]
    noise = (msum < 1.0).astype(jnp.float32)
    sel = 1.0 - noise
    qq = q - Q_MIN

    selq = sel * qq
    temp_div = jnp.sum(selq)
    temp_loss = jnp.sum(selq * loss_elem)
    gen = jnp.where(temp_div == 0.0, 0.0,
                    temp_loss / jnp.where(temp_div == 0.0, 1.0, temp_div))

    n_b = jnp.sum(noise)
    bg_noise = SUPRESSION * jnp.sum(noise * beta) / n_b

    lin_i = (jax.lax.broadcasted_iota(jnp.int32, (_H, _W), 0) * _W
             + jax.lax.broadcasted_iota(jnp.int32, (_H, _W), 1))
    lin = lin_i.astype(jnp.float32)
    temp = jnp.zeros_like(q)
    ba_sum = 0.0
    for i in range(_NOBJ):
        m = m_ref[0, i]
        ba_sum = ba_sum + jnp.max(beta * m)
        qm = q * m
        qa = jnp.max(qm)
        idx = jnp.min(jnp.where(qm == qa, lin, jnp.float32(_N)))
        onehot = (lin == idx).astype(jnp.float32)
        a0 = jnp.sum(onehot * x0)
        a1 = jnp.sum(onehot * x1)
        d0 = x0 - a0
        d1 = x1 - a1
        n2 = d0 * d0 + d1 * d1
        xn = jnp.sqrt(n2)
        att = xn * xn * qa
        rep = jnp.maximum(0.0, 1.0 - xn) * qa
        temp = temp + m * att + (1.0 - m) * rep
    pot = jnp.sum(q * temp) * (1.0 / _N)
    bg = (1.0 - ba_sum * (1.0 / _NOBJ)) + bg_noise

    loss = gen + COND_WEIGHT * (bg + pot)
    out_ref[...] = jnp.full((1, 1, 128), loss, dtype=jnp.float32)


def kernel(x, beta, matrix, input, target):
    B, n_objects, H, W = matrix.shape
    x0 = x[..., 0]
    x1 = x[..., 1]

    out = pl.pallas_call(
        _loss_kernel,
        grid=(B,),
        in_specs=[
            pl.BlockSpec((1, H, W), lambda b: (b, 0, 0)),
            pl.BlockSpec((1, H, W), lambda b: (b, 0, 0)),
            pl.BlockSpec((1, H, W), lambda b: (b, 0, 0)),
            pl.BlockSpec((1, n_objects, H, W), lambda b: (b, 0, 0, 0)),
            pl.BlockSpec((1, 1, H, W), lambda b: (b, 0, 0, 0)),
            pl.BlockSpec((1, 1, H, W), lambda b: (b, 0, 0, 0)),
        ],
        out_specs=pl.BlockSpec((1, 1, 128), lambda b: (b, 0, 0)),
        out_shape=jax.ShapeDtypeStruct((B, 1, 128), jnp.float32),
        compiler_params=pltpu.CompilerParams(
            dimension_semantics=("parallel",)),
    )(x0, x1, beta, matrix, input, target)
    return jnp.mean(out[:, 0, 0])


# speedup vs baseline: 4.8932x; 1.6852x over previous
import jax
import jax.numpy as jnp
from jax.experimental import pallas as pl
from jax.experimental.pallas import tpu as pltpu

Q_MIN = 0.1
SUPRESSION = 0.1
COND_WEIGHT = 1.0

_B, _NOBJ, _H, _W = 8, 8, 128, 128
_N = _H * _W


def _atanh(v):
    return jnp.log1p(2.0 * v / (1.0 - v)) / 2.0


def _loss_kernel(x0_ref, x1_ref, beta_ref, m_ref, inp_ref, tgt_ref, out_ref):
    beta = beta_ref[0]
    x0 = x0_ref[0]
    x1 = x1_ref[0]
    inp = inp_ref[0, 0]
    tgt = tgt_ref[0, 0]
    m3 = m_ref[0]

    q = _atanh(beta) ** 2 + Q_MIN
    loss_elem = (inp - tgt) ** 2

    msum = jnp.sum(m3, axis=0)
    noise = (msum < 1.0).astype(jnp.float32)
    sel = 1.0 - noise
    qq = q - Q_MIN

    selq = sel * qq
    temp_div = jnp.sum(selq)
    temp_loss = jnp.sum(selq * loss_elem)
    gen = jnp.where(temp_div == 0.0, 0.0,
                    temp_loss / jnp.where(temp_div == 0.0, 1.0, temp_div))

    n_b = jnp.sum(noise)
    bg_noise = SUPRESSION * jnp.sum(noise * beta) / n_b
    ba3 = jnp.max(beta[None] * m3, axis=(1, 2))

    lin = (jax.lax.broadcasted_iota(jnp.int32, (_H, _W), 0) * _W
           + jax.lax.broadcasted_iota(jnp.int32, (_H, _W), 1)
           ).astype(jnp.float32)
    qm3 = q[None] * m3
    qa3 = jnp.max(qm3, axis=(1, 2), keepdims=True)
    idx3 = jnp.min(jnp.where(qm3 == qa3, lin[None], jnp.float32(_N)),
                   axis=(1, 2), keepdims=True)
    oh3 = (lin[None] == idx3).astype(jnp.float32)
    a03 = jnp.sum(oh3 * x0[None], axis=(1, 2), keepdims=True)
    a13 = jnp.sum(oh3 * x1[None], axis=(1, 2), keepdims=True)

    d0 = x0[None] - a03
    d1 = x1[None] - a13
    n2 = d0 * d0 + d1 * d1
    xn = jnp.sqrt(n2)
    rep = jnp.maximum(1.0 - xn, 0.0)
    blend = rep + m3 * (n2 - rep)
    temp = jnp.sum(qa3 * blend, axis=0)
    pot = jnp.sum(q * temp) * (1.0 / _N)

    bg = (1.0 - jnp.sum(ba3) * (1.0 / _NOBJ)) + bg_noise
    loss = gen + COND_WEIGHT * (bg + pot)
    out_ref[...] = jnp.full((1, 1, 128), loss, dtype=jnp.float32)


def kernel(x, beta, matrix, input, target):
    B, n_objects, H, W = matrix.shape
    x0 = x[..., 0]
    x1 = x[..., 1]

    out = pl.pallas_call(
        _loss_kernel,
        grid=(B,),
        in_specs=[
            pl.BlockSpec((1, H, W), lambda b: (b, 0, 0)),
            pl.BlockSpec((1, H, W), lambda b: (b, 0, 0)),
            pl.BlockSpec((1, H, W), lambda b: (b, 0, 0)),
            pl.BlockSpec((1, n_objects, H, W), lambda b: (b, 0, 0, 0)),
            pl.BlockSpec((1, 1, H, W), lambda b: (b, 0, 0, 0)),
            pl.BlockSpec((1, 1, H, W), lambda b: (b, 0, 0, 0)),
        ],
        out_specs=pl.BlockSpec((1, 1, 128), lambda b: (b, 0, 0)),
        out_shape=jax.ShapeDtypeStruct((B, 1, 128), jnp.float32),
        compiler_params=pltpu.CompilerParams(
            dimension_semantics=("parallel",)),
    )(x0, x1, beta, matrix, input, target)
    return jnp.mean(out[:, 0, 0])


# speedup vs baseline: 5.7093x; 1.1668x over previous
import jax
import jax.numpy as jnp
from jax.experimental import pallas as pl
from jax.experimental.pallas import tpu as pltpu

Q_MIN = 0.1
SUPRESSION = 0.1
COND_WEIGHT = 1.0

_B, _NOBJ, _H, _W = 8, 8, 128, 128
_N = _H * _W


def _atanh(v):
    return jnp.log1p(2.0 * v / (1.0 - v)) / 2.0


def _loss_kernel(x0_ref, x1_ref, beta_ref, m_ref, inp_ref, tgt_ref, out_ref):
    beta = beta_ref[0]
    x0 = x0_ref[0]
    x1 = x1_ref[0]
    inp = inp_ref[0, 0]
    tgt = tgt_ref[0, 0]
    m3 = m_ref[0]

    q = _atanh(beta) ** 2 + Q_MIN
    loss_elem = (inp - tgt) ** 2

    msum = jnp.sum(m3, axis=0)
    noise = (msum < 1.0).astype(jnp.float32)
    sel = 1.0 - noise
    qq = q - Q_MIN

    selq = sel * qq
    temp_div = jnp.sum(selq)
    temp_loss = jnp.sum(selq * loss_elem)
    gen = jnp.where(temp_div == 0.0, 0.0,
                    temp_loss / jnp.where(temp_div == 0.0, 1.0, temp_div))

    n_b = jnp.sum(noise)
    bg_noise = SUPRESSION * jnp.sum(noise * beta) / n_b
    ba3 = jnp.max(beta[None] * m3, axis=(1, 2))

    lin = (jax.lax.broadcasted_iota(jnp.int32, (_H, _W), 0) * _W
           + jax.lax.broadcasted_iota(jnp.int32, (_H, _W), 1)
           ).astype(jnp.float32)
    qm3 = q[None] * m3
    qa3 = jnp.max(qm3, axis=(1, 2), keepdims=True)
    idx3 = jnp.min(jnp.where(qm3 == qa3, lin[None], jnp.float32(_N)),
                   axis=(1, 2), keepdims=True)
    oh3 = (lin[None] == idx3).astype(jnp.float32)
    a03 = jnp.sum(oh3 * x0[None], axis=(1, 2), keepdims=True)
    a13 = jnp.sum(oh3 * x1[None], axis=(1, 2), keepdims=True)

    d0 = x0[None] - a03
    d1 = x1[None] - a13
    n2 = d0 * d0 + d1 * d1
    xn = jnp.sqrt(n2)
    rep = jnp.maximum(1.0 - xn, 0.0)
    blend = rep + m3 * (n2 - rep)
    temp = jnp.sum(qa3 * blend, axis=0)
    pot = jnp.sum(q * temp) * (1.0 / _N)

    bg = (1.0 - jnp.sum(ba3) * (1.0 / _NOBJ)) + bg_noise
    loss = gen + COND_WEIGHT * (bg + pot)

    pid = pl.program_id(0)

    @pl.when(pid == 0)
    def _init():
        out_ref[...] = jnp.zeros((1, 1), jnp.float32)

    out_ref[...] += jnp.full((1, 1), loss * (1.0 / _B), jnp.float32)


def kernel(x, beta, matrix, input, target):
    B, n_objects, H, W = matrix.shape
    x0 = x[..., 0]
    x1 = x[..., 1]

    out = pl.pallas_call(
        _loss_kernel,
        grid=(B,),
        in_specs=[
            pl.BlockSpec((1, H, W), lambda b: (b, 0, 0)),
            pl.BlockSpec((1, H, W), lambda b: (b, 0, 0)),
            pl.BlockSpec((1, H, W), lambda b: (b, 0, 0)),
            pl.BlockSpec((1, n_objects, H, W), lambda b: (b, 0, 0, 0)),
            pl.BlockSpec((1, 1, H, W), lambda b: (b, 0, 0, 0)),
            pl.BlockSpec((1, 1, H, W), lambda b: (b, 0, 0, 0)),
        ],
        out_specs=pl.BlockSpec((1, 1), lambda b: (0, 0)),
        out_shape=jax.ShapeDtypeStruct((1, 1), jnp.float32),
        compiler_params=pltpu.CompilerParams(
            dimension_semantics=("arbitrary",)),
    )(x0, x1, beta, matrix, input, target)
    return out.reshape(())


# speedup vs baseline: 6.1550x; 1.0781x over previous
import jax
import jax.numpy as jnp
from jax.experimental import pallas as pl
from jax.experimental.pallas import tpu as pltpu

Q_MIN = 0.1
SUPRESSION = 0.1
COND_WEIGHT = 1.0

_B, _NOBJ, _H, _W = 8, 8, 128, 128
_N = _H * _W
_IPS = 2


def _atanh(v):
    return jnp.log1p(2.0 * v / (1.0 - v)) / 2.0


def _image_loss(x0, x1, beta, m3, inp, tgt, lin):
    q = _atanh(beta) ** 2 + Q_MIN
    loss_elem = (inp - tgt) ** 2

    msum = jnp.sum(m3, axis=0)
    noise = (msum < 1.0).astype(jnp.float32)
    sel = 1.0 - noise
    qq = q - Q_MIN

    selq = sel * qq
    temp_div = jnp.sum(selq)
    temp_loss = jnp.sum(selq * loss_elem)
    gen = jnp.where(temp_div == 0.0, 0.0,
                    temp_loss / jnp.where(temp_div == 0.0, 1.0, temp_div))

    n_b = jnp.sum(noise)
    bg_noise = SUPRESSION * jnp.sum(noise * beta) / n_b
    ba3 = jnp.max(beta[None] * m3, axis=(1, 2))

    qm3 = q[None] * m3
    qa3 = jnp.max(qm3, axis=(1, 2), keepdims=True)
    idx3 = jnp.min(jnp.where(qm3 == qa3, lin[None], jnp.float32(_N)),
                   axis=(1, 2), keepdims=True)
    oh3 = (lin[None] == idx3).astype(jnp.float32)
    a03 = jnp.sum(oh3 * x0[None], axis=(1, 2), keepdims=True)
    a13 = jnp.sum(oh3 * x1[None], axis=(1, 2), keepdims=True)

    d0 = x0[None] - a03
    d1 = x1[None] - a13
    n2 = d0 * d0 + d1 * d1
    xn = jnp.sqrt(n2)
    rep = jnp.maximum(1.0 - xn, 0.0)
    blend = rep + m3 * (n2 - rep)
    temp = jnp.sum(qa3 * blend, axis=0)
    pot = jnp.sum(q * temp) * (1.0 / _N)

    bg = (1.0 - jnp.sum(ba3) * (1.0 / _NOBJ)) + bg_noise
    return gen + COND_WEIGHT * (bg + pot)


def _loss_kernel(x0_ref, x1_ref, beta_ref, m_ref, inp_ref, tgt_ref, out_ref):
    lin = (jax.lax.broadcasted_iota(jnp.int32, (_H, _W), 0) * _W
           + jax.lax.broadcasted_iota(jnp.int32, (_H, _W), 1)
           ).astype(jnp.float32)

    acc = 0.0
    for j in range(_IPS):
        acc += _image_loss(x0_ref[j], x1_ref[j], beta_ref[j], m_ref[j],
                           inp_ref[j, 0], tgt_ref[j, 0], lin)

    pid = pl.program_id(0)

    @pl.when(pid == 0)
    def _init():
        out_ref[...] = jnp.zeros((1, 1), jnp.float32)

    out_ref[...] += jnp.full((1, 1), acc * (1.0 / _B), jnp.float32)


def kernel(x, beta, matrix, input, target):
    B, n_objects, H, W = matrix.shape
    x0 = x[..., 0]
    x1 = x[..., 1]

    out = pl.pallas_call(
        _loss_kernel,
        grid=(B // _IPS,),
        in_specs=[
            pl.BlockSpec((_IPS, H, W), lambda b: (b, 0, 0)),
            pl.BlockSpec((_IPS, H, W), lambda b: (b, 0, 0)),
            pl.BlockSpec((_IPS, H, W), lambda b: (b, 0, 0)),
            pl.BlockSpec((_IPS, n_objects, H, W), lambda b: (b, 0, 0, 0)),
            pl.BlockSpec((_IPS, 1, H, W), lambda b: (b, 0, 0, 0)),
            pl.BlockSpec((_IPS, 1, H, W), lambda b: (b, 0, 0, 0)),
        ],
        out_specs=pl.BlockSpec((1, 1), lambda b: (0, 0)),
        out_shape=jax.ShapeDtypeStruct((1, 1), jnp.float32),
        compiler_params=pltpu.CompilerParams(
            dimension_semantics=("arbitrary",)),
    )(x0, x1, beta, matrix, input, target)
    return out.reshape(())
